# Initial kernel scaffold; baseline (speedup 1.0000x reference)
import jax
import jax.numpy as jnp
from jax.experimental import pallas as pl


def kernel(word_u, word_v, in_embed_w, in_bias_w, out_embed_w, out_bias_w):
    raise NotImplementedError("write your pallas kernel here")


# speedup vs baseline: 3.2046x; 3.2046x over previous
import jax
import jax.numpy as jnp
from jax.experimental import pallas as pl
from jax.experimental.pallas import tpu as pltpu


def _tc_kernel(wu_ref, wv_ref, ie_ref, ib_ref, oe_ref, ob_ref, o_ref,
               s_ref, b_ref):
    i = pl.program_id(0)
    B = wu_ref.shape[0]
    V, D = ie_ref.shape
    TI = o_ref.shape[0]

    @pl.when(i == 0)
    def _():
        iot = jax.lax.broadcasted_iota(jnp.int32, (B, V), 1)
        ohu = (wu_ref[...] == iot).astype(jnp.float32)
        ohv = (wv_ref[...] == iot).astype(jnp.float32)
        ue = jnp.dot(ohu, ie_ref[...], preferred_element_type=jnp.float32)
        ve = jnp.dot(ohv, oe_ref[...], preferred_element_type=jnp.float32)
        prod = ue * ve
        ones = jnp.ones((1, D), jnp.float32)
        s_ref[...] = jax.lax.dot_general(
            ones, prod, (((1,), (1,)), ((), ())),
            preferred_element_type=jnp.float32)
        b_ref[...] = (
            jnp.dot(ohu, ib_ref[...], preferred_element_type=jnp.float32)
            + jnp.dot(ohv, ob_ref[...], preferred_element_type=jnp.float32))

    o_ref[...] = b_ref[pl.ds(i * TI, TI), :] + s_ref[...]


def kernel(word_u, word_v, in_embed_w, in_bias_w, out_embed_w, out_bias_w):
    B = word_u.shape[0]
    V, D = in_embed_w.shape
    TI = 512
    wu = word_u.astype(jnp.int32).reshape(B, 1)
    wv = word_v.astype(jnp.int32).reshape(B, 1)

    return pl.pallas_call(
        _tc_kernel,
        grid=(B // TI,),
        in_specs=[
            pl.BlockSpec((B, 1), lambda i: (0, 0)),
            pl.BlockSpec((B, 1), lambda i: (0, 0)),
            pl.BlockSpec((V, D), lambda i: (0, 0)),
            pl.BlockSpec((V, 1), lambda i: (0, 0)),
            pl.BlockSpec((V, D), lambda i: (0, 0)),
            pl.BlockSpec((V, 1), lambda i: (0, 0)),
        ],
        out_specs=pl.BlockSpec((TI, B), lambda i: (i, 0)),
        out_shape=jax.ShapeDtypeStruct((B, B), jnp.float32),
        scratch_shapes=[
            pltpu.VMEM((1, B), jnp.float32),
            pltpu.VMEM((B, 1), jnp.float32),
        ],
    )(wu, wv, in_embed_w, in_bias_w, out_embed_w, out_bias_w)
